# baseline (speedup 1.0000x reference)
import jax
import jax.numpy as jnp
from jax.experimental import pallas as pl

_B = 16
_D = 64
_HW = 1024
_K = 1024
_COMMIT = 0.25


_IMGS = 4
_C = 128


def _vq_body(x_ref, e_ref, out_ref, idx_ref, loss_ref):
    e = e_ref[...]
    esq = jnp.sum(e * e, axis=1)
    e2 = e + e
    riota = jax.lax.broadcasted_iota(jnp.int32, (_K, 1), 0)
    hicol = (riota // 32).astype(jnp.float32)
    locol = (riota % 32).astype(jnp.float32)
    eaug = jnp.concatenate([e, hicol, locol], axis=1)
    loss = jnp.zeros((), jnp.float32)
    for k in range(_IMGS):
        xb = x_ref[k]
        xt = xb.T
        xsq = jnp.sum(xt * xt, axis=1, keepdims=True)
        mm2 = jax.lax.dot_general(xt, e2, (((1,), (1,)), ((), ())))
        val = (xsq + esq[0:_C]) - mm2[:, 0:_C]
        gch = jnp.zeros((_HW, _C), jnp.int32)
        for c in range(1, _K // _C):
            d_c = (xsq + esq[c * _C:(c + 1) * _C]) - mm2[:, c * _C:(c + 1) * _C]
            take = d_c < val
            val = jnp.where(take, d_c, val)
            gch = jnp.where(take, c, gch)
        minv = jnp.min(val, axis=1, keepdims=True)
        lane = jax.lax.broadcasted_iota(jnp.int32, (_HW, _C), 1)
        cand = jnp.where(val == minv, gch * _C + lane, _K)
        candm = jnp.min(cand, axis=1, keepdims=True)
        win = cand == candm
        ohfull = jnp.concatenate(
            [jnp.where(win & (gch == c), 1.0, 0.0)
             for c in range(_K // _C)], axis=1)
        q66 = jax.lax.dot_general(
            eaug, ohfull, (((0,), (1,)), ((), ())))
        q_t = q66[0:_D, :]
        tail = q66[_D:_D + 2, :]
        idx_ref[k, 0, :] = (tail[0].astype(jnp.int32) * 32
                            + tail[1].astype(jnp.int32))
        diff = q_t - xb
        out_ref[k] = xb + diff
        loss = loss + jnp.sum(diff * diff)
    loss_ref[...] = loss.reshape(1, 1, 1)


def kernel(x, embeddings):
    x3 = x.reshape(_B, _D, _HW)
    out, idx, loss = pl.pallas_call(
        _vq_body,
        grid=(_B // _IMGS,),
        in_specs=[
            pl.BlockSpec((_IMGS, _D, _HW), lambda i: (i, 0, 0)),
            pl.BlockSpec((_K, _D), lambda i: (0, 0)),
        ],
        out_specs=[
            pl.BlockSpec((_IMGS, _D, _HW), lambda i: (i, 0, 0)),
            pl.BlockSpec((_IMGS, 1, _HW), lambda i: (i, 0, 0)),
            pl.BlockSpec((1, 1, 1), lambda i: (i, 0, 0)),
        ],
        out_shape=[
            jax.ShapeDtypeStruct((_B, _D, _HW), jnp.float32),
            jax.ShapeDtypeStruct((_B, 1, _HW), jnp.int32),
            jax.ShapeDtypeStruct((_B // _IMGS, 1, 1), jnp.float32),
        ],
    )(x3, embeddings)
    out4 = out.reshape(x.shape)
    enc = idx.reshape(_B, _HW)
    d = jnp.sum(loss) / (_B * _D * _HW)
    total_loss = d + _COMMIT * d
    return out4, total_loss, enc, embeddings


# speedup vs baseline: 1.0268x; 1.0268x over previous
import jax
import jax.numpy as jnp
from jax.experimental import pallas as pl

_B = 16
_D = 64
_HW = 1024
_K = 1024
_COMMIT = 0.25


_IMGS = 4
_C = 128


def _vq_body(x_ref, e_ref, out_ref, idx_ref, loss_ref):
    e = e_ref[...]
    esq = jnp.sum(e * e, axis=1)
    e2 = e + e
    loss = jnp.zeros((), jnp.float32)
    for k in range(_IMGS):
        xb = x_ref[k]
        xt = xb.T
        xsq = jnp.sum(xt * xt, axis=1, keepdims=True)
        mm2 = jax.lax.dot_general(xt, e2, (((1,), (1,)), ((), ())))
        xsqb = jnp.broadcast_to(xsq, (_HW, _C))
        val = (xsqb + esq[0:_C]) - mm2[:, 0:_C]
        gch = jnp.zeros((_HW, _C), jnp.int32)
        for c in range(1, _K // _C):
            d_c = (xsqb + esq[c * _C:(c + 1) * _C]) - mm2[:, c * _C:(c + 1) * _C]
            take = d_c < val
            val = jnp.where(take, d_c, val)
            gch = jnp.where(take, c, gch)
        minv = jnp.min(val, axis=1, keepdims=True)
        lane = jax.lax.broadcasted_iota(jnp.int32, (_HW, _C), 1)
        cand = jnp.where(val == minv, gch * _C + lane, _K)
        idx = jnp.min(cand, axis=1).astype(jnp.int32)
        idx_ref[k, 0, :] = idx
        iota = jax.lax.broadcasted_iota(jnp.int32, (_HW, _K), 1)
        onehot = (idx[:, None] == iota).astype(jnp.float32)
        q_t = jax.lax.dot_general(e, onehot, (((0,), (1,)), ((), ())))
        diff = q_t - xb
        out_ref[k] = xb + diff
        loss = loss + jnp.sum(diff * diff)
    loss_ref[...] = loss.reshape(1, 1, 1)


def kernel(x, embeddings):
    x3 = x.reshape(_B, _D, _HW)
    out, idx, loss = pl.pallas_call(
        _vq_body,
        grid=(_B // _IMGS,),
        in_specs=[
            pl.BlockSpec((_IMGS, _D, _HW), lambda i: (i, 0, 0)),
            pl.BlockSpec((_K, _D), lambda i: (0, 0)),
        ],
        out_specs=[
            pl.BlockSpec((_IMGS, _D, _HW), lambda i: (i, 0, 0)),
            pl.BlockSpec((_IMGS, 1, _HW), lambda i: (i, 0, 0)),
            pl.BlockSpec((1, 1, 1), lambda i: (i, 0, 0)),
        ],
        out_shape=[
            jax.ShapeDtypeStruct((_B, _D, _HW), jnp.float32),
            jax.ShapeDtypeStruct((_B, 1, _HW), jnp.int32),
            jax.ShapeDtypeStruct((_B // _IMGS, 1, 1), jnp.float32),
        ],
    )(x3, embeddings)
    out4 = out.reshape(x.shape)
    enc = idx.reshape(_B, _HW)
    d = jnp.sum(loss) / (_B * _D * _HW)
    total_loss = d + _COMMIT * d
    return out4, total_loss, enc, embeddings
